# Initial kernel scaffold; baseline (speedup 1.0000x reference)
import jax
import jax.numpy as jnp
from jax.experimental import pallas as pl


def kernel(features, rows0, cols0, data0, kernel0, rows1, cols1, data1, kernel1, bias):
    raise NotImplementedError("write your pallas kernel here")


# speedup vs baseline: 2.2097x; 2.2097x over previous
import dataclasses

import jax
import jax.numpy as jnp
from jax import lax
from jax.experimental import pallas as pl
from jax.experimental.pallas import tpu as pltpu
from jax.experimental.pallas import tpu_sc as plsc

N = 10000
B = 128
NPAD = 10240
NTILES = 16
NCORES = 2
R = NPAD // NTILES
C = 128
L = 16
EP = 320000 + C


def _scalar_from(ref, idx):
    v = plsc.load_gather(ref, [jnp.full((L,), idx, jnp.int32)])
    return jnp.max(v)


def _sc_body(f_hbm, rows_hbm, cols_hbm, data_hbm, kern_hbm, meta_hbm, out_hbm,
             meta_v, cols_v, rows_v, data_v, kern_v, vals_v, rl_v, g_v, acc_v,
             sem):
    c = lax.axis_index("c")
    s = lax.axis_index("s")
    wid = c * NTILES + s
    base = s * R

    pltpu.sync_copy(meta_hbm, meta_v)
    start = _scalar_from(meta_v, wid)
    nch = _scalar_from(meta_v, wid + 32)

    iota = lax.iota(jnp.int32, L)
    col_idx = [iota + j * L for j in range(B // L)]
    zeros = jnp.zeros((L,), jnp.float32)

    @pl.loop(0, R)
    def _(r):
        rfull = jnp.full((L,), r, jnp.int32)
        for j in range(B // L):
            plsc.store_scatter(acc_v, [rfull, col_idx[j]], zeros)

    ebase = c * EP

    @pl.loop(0, nch)
    def _(k):
        off = pl.multiple_of(ebase + start + k * C, 8)
        pltpu.sync_copy(cols_hbm.at[pl.ds(off, C)], cols_v)
        gather = pltpu.async_copy(f_hbm.at[cols_v], g_v, sem)
        pltpu.sync_copy(rows_hbm.at[pl.ds(off, C)], rows_v)
        pltpu.sync_copy(data_hbm.at[pl.ds(off, C)], data_v)
        pltpu.sync_copy(kern_hbm.at[pl.ds(off, C)], kern_v)

        for g in range(C // L):
            sl = pl.ds(g * L, L)
            r = rows_v[sl]
            val = data_v[sl] * kern_v[sl]
            inr = (r >= base) & (r < base + R)
            vals_v[sl] = jnp.where(inr, val, 0.0)
            rl_v[sl] = jnp.clip(r - base, 0, R - 1)
        gather.wait()

        @pl.loop(0, C)
        def _(i):
            ii = jnp.full((L,), i, jnp.int32)
            vspl = plsc.load_gather(vals_v, [ii])
            rspl = plsc.load_gather(rl_v, [ii])
            for j in range(B // L):
                gv = plsc.load_gather(g_v, [ii, col_idx[j]])
                plsc.addupdate_scatter(acc_v, [rspl, col_idx[j]], gv * vspl)

    obase = pl.multiple_of(c * NPAD + base, 8)
    pltpu.sync_copy(acc_v, out_hbm.at[pl.ds(obase, R)])


def _sc_spmm(f, rows_s, cols_s, data_s, kern_s, meta):
    mesh = plsc.VectorSubcoreMesh(core_axis_name="c", subcore_axis_name="s")
    cp = pltpu.CompilerParams()
    if "needs_layout_passes" in pltpu.CompilerParams.__dataclass_fields__:
        cp = dataclasses.replace(cp, needs_layout_passes=False)
    kfn = pl.kernel(
        _sc_body,
        compiler_params=cp,
        out_type=jax.ShapeDtypeStruct((NCORES * NPAD, B), jnp.float32),
        mesh=mesh,
        scratch_types=[
            pltpu.VMEM((64,), jnp.int32),
            pltpu.VMEM((C,), jnp.int32),
            pltpu.VMEM((C,), jnp.int32),
            pltpu.VMEM((C,), jnp.float32),
            pltpu.VMEM((C,), jnp.float32),
            pltpu.VMEM((C,), jnp.float32),
            pltpu.VMEM((C,), jnp.int32),
            pltpu.VMEM((C, B), jnp.float32),
            pltpu.VMEM((R, B), jnp.float32),
            pltpu.SemaphoreType.DMA,
        ],
    )
    return kfn(f, rows_s, cols_s, data_s, kern_s, meta)


def _combine_body(p_ref, b_ref, o_ref):
    t = p_ref[0] + p_ref[1]
    o_ref[...] = jnp.maximum(t.T + b_ref[0], 0.0)


def _tc_combine(partials, bias2d):
    return pl.pallas_call(
        _combine_body,
        grid=(NPAD // 128,),
        in_specs=[
            pl.BlockSpec((2, 128, 128), lambda i: (0, i, 0)),
            pl.BlockSpec((1, 1, 128), lambda i: (i, 0, 0)),
        ],
        out_specs=pl.BlockSpec((128, 128), lambda i: (0, i)),
        out_shape=jax.ShapeDtypeStruct((B, NPAD), jnp.float32),
    )(partials, bias2d)


def kernel(features, rows0, cols0, data0, kernel0, rows1, cols1, data1,
           kernel1, bias):
    f = features.T

    bounds = jnp.arange(NTILES + 1, dtype=jnp.int32) * R

    def prep(rows):
        ss = jnp.searchsorted(rows, bounds).astype(jnp.int32)
        st = ss[:NTILES]
        en = ss[1:]
        st_al = (st // 8) * 8
        nch = (en - st_al + C - 1) // C
        return st_al, nch

    s0, n0 = prep(rows0)
    s1, n1 = prep(rows1)
    meta = jnp.concatenate([s0, s1, n0, n1]).astype(jnp.int32)

    padi = jnp.full((C,), NPAD, jnp.int32)
    padz = jnp.zeros((C,), jnp.float32)
    pad0 = jnp.zeros((C,), jnp.int32)
    rows_s = jnp.concatenate([rows0, padi, rows1, padi])
    cols_s = jnp.concatenate([cols0, pad0, cols1, pad0])
    data_s = jnp.concatenate([data0, padz, data1, padz])
    kern_s = jnp.concatenate([kernel0, padz, kernel1, padz])

    partials = _sc_spmm(f, rows_s, cols_s, data_s, kern_s, meta)
    partials = partials.reshape(NCORES, NPAD, B)

    biasp = jnp.concatenate([bias, jnp.zeros((NPAD - N,), jnp.float32)])
    bias2d = biasp.reshape(NPAD // 128, 1, 128)

    out_full = _tc_combine(partials, bias2d)
    return out_full[:, :N]


# speedup vs baseline: 2.8228x; 1.2774x over previous
import dataclasses

import jax
import jax.numpy as jnp
from jax import lax
from jax.experimental import pallas as pl
from jax.experimental.pallas import tpu as pltpu
from jax.experimental.pallas import tpu_sc as plsc

N = 10000
B = 128
NPAD = 10240
NTILES = 16
NCORES = 2
R = NPAD // NTILES
C = 128
L = 16
EP = 320000 + 4 * C


def _scalar_from(ref, idx):
    v = plsc.load_gather(ref, [jnp.full((L,), idx, jnp.int32)])
    return jnp.max(v)


def _sc_body(f_hbm, rows_hbm, cols_hbm, data_hbm, kern_hbm, meta_hbm, out_hbm,
             meta_v, cols0_v, rows0_v, data0_v, kern0_v,
             cols1_v, rows1_v, data1_v, kern1_v,
             vals_v, rl_v, g0_v, g1_v, acc_v, se0, se1, sg0, sg1):
    c = lax.axis_index("c")
    s = lax.axis_index("s")
    wid = c * NTILES + s
    base = s * R

    pltpu.sync_copy(meta_hbm, meta_v)
    start = _scalar_from(meta_v, wid)
    npairs = _scalar_from(meta_v, wid + 32)

    iota = lax.iota(jnp.int32, L)
    col_idx = [iota + j * L for j in range(B // L)]
    zeros = jnp.zeros((L,), jnp.float32)

    @pl.loop(0, R)
    def _(r):
        rfull = jnp.full((L,), r, jnp.int32)
        for j in range(B // L):
            plsc.store_scatter(acc_v, [rfull, col_idx[j]], zeros)

    ebase = c * EP
    bufs = ((cols0_v, rows0_v, data0_v, kern0_v, g0_v, se0, sg0),
            (cols1_v, rows1_v, data1_v, kern1_v, g1_v, se1, sg1))

    def edges_issue(n, half):
        cv, rv, dv, kv, _, se, _ = bufs[half]
        off = pl.multiple_of(ebase + start + n * C, 8)
        pltpu.async_copy(cols_hbm.at[pl.ds(off, C)], cv, se)
        pltpu.async_copy(rows_hbm.at[pl.ds(off, C)], rv, se)
        pltpu.async_copy(data_hbm.at[pl.ds(off, C)], dv, se)
        pltpu.async_copy(kern_hbm.at[pl.ds(off, C)], kv, se)

    def edges_drain(half):
        cv, rv, dv, kv, _, se, _ = bufs[half]
        pltpu.make_async_copy(cols_hbm.at[pl.ds(0, C)], cv, se).wait()
        pltpu.make_async_copy(rows_hbm.at[pl.ds(0, C)], rv, se).wait()
        pltpu.make_async_copy(data_hbm.at[pl.ds(0, C)], dv, se).wait()
        pltpu.make_async_copy(kern_hbm.at[pl.ds(0, C)], kv, se).wait()

    def gather_issue(half):
        cv, _, _, _, gv, _, sg = bufs[half]
        pltpu.async_copy(f_hbm.at[cv], gv, sg)

    def gather_drain(half):
        cv, _, _, _, gv, _, sg = bufs[half]
        pltpu.make_async_copy(f_hbm.at[cv], gv, sg).wait()

    edges_issue(0, 0)
    edges_drain(0)
    gather_issue(0)
    edges_issue(1, 1)

    def process(n, half):
        cv, rv, dv, kv, gv, _, _ = bufs[half]
        other = 1 - half
        edges_drain(other)
        gather_issue(other)
        for g in range(C // L):
            sl = pl.ds(g * L, L)
            r = rv[sl]
            val = dv[sl] * kv[sl]
            inr = (r >= base) & (r < base + R)
            vals_v[sl] = jnp.where(inr, val, 0.0)
            rl_v[sl] = jnp.clip(r - base, 0, R - 1)
        gather_drain(half)
        edges_issue(n + 2, half)
        @pl.loop(0, C)
        def _(i):
            ii = jnp.full((L,), i, jnp.int32)
            vspl = plsc.load_gather(vals_v, [ii])
            rspl = plsc.load_gather(rl_v, [ii])
            for j in range(B // L):
                gvv = plsc.load_gather(gv, [ii, col_idx[j]])
                plsc.addupdate_scatter(acc_v, [rspl, col_idx[j]], gvv * vspl)

    @pl.loop(0, npairs)
    def _(p):
        process(2 * p, 0)
        process(2 * p + 1, 1)

    gather_drain(0)
    edges_drain(1)

    obase = pl.multiple_of(c * NPAD + base, 8)
    pltpu.sync_copy(acc_v, out_hbm.at[pl.ds(obase, R)])


def _sc_spmm(f, rows_s, cols_s, data_s, kern_s, meta):
    mesh = plsc.VectorSubcoreMesh(core_axis_name="c", subcore_axis_name="s")
    cp = pltpu.CompilerParams()
    if "needs_layout_passes" in pltpu.CompilerParams.__dataclass_fields__:
        cp = dataclasses.replace(cp, needs_layout_passes=False)
    kfn = pl.kernel(
        _sc_body,
        compiler_params=cp,
        out_type=jax.ShapeDtypeStruct((NCORES * NPAD, B), jnp.float32),
        mesh=mesh,
        scratch_types=[
            pltpu.VMEM((64,), jnp.int32),
            pltpu.VMEM((C,), jnp.int32),
            pltpu.VMEM((C,), jnp.int32),
            pltpu.VMEM((C,), jnp.float32),
            pltpu.VMEM((C,), jnp.float32),
            pltpu.VMEM((C,), jnp.int32),
            pltpu.VMEM((C,), jnp.int32),
            pltpu.VMEM((C,), jnp.float32),
            pltpu.VMEM((C,), jnp.float32),
            pltpu.VMEM((C,), jnp.float32),
            pltpu.VMEM((C,), jnp.int32),
            pltpu.VMEM((C, B), jnp.float32),
            pltpu.VMEM((C, B), jnp.float32),
            pltpu.VMEM((R, B), jnp.float32),
            pltpu.SemaphoreType.DMA,
            pltpu.SemaphoreType.DMA,
            pltpu.SemaphoreType.DMA,
            pltpu.SemaphoreType.DMA,
        ],
    )
    return kfn(f, rows_s, cols_s, data_s, kern_s, meta)


def _combine_body(p_ref, b_ref, o_ref):
    t = p_ref[0] + p_ref[1]
    o_ref[...] = jnp.maximum(t.T + b_ref[0], 0.0)


def _tc_combine(partials, bias2d):
    return pl.pallas_call(
        _combine_body,
        grid=(NPAD // 128,),
        in_specs=[
            pl.BlockSpec((2, 128, 128), lambda i: (0, i, 0)),
            pl.BlockSpec((1, 1, 128), lambda i: (i, 0, 0)),
        ],
        out_specs=pl.BlockSpec((128, 128), lambda i: (0, i)),
        out_shape=jax.ShapeDtypeStruct((B, NPAD), jnp.float32),
    )(partials, bias2d)


def kernel(features, rows0, cols0, data0, kernel0, rows1, cols1, data1,
           kernel1, bias):
    f = features.T

    bounds = jnp.arange(NTILES + 1, dtype=jnp.int32) * R

    def prep(rows):
        ss = jnp.searchsorted(rows, bounds).astype(jnp.int32)
        st = ss[:NTILES]
        en = ss[1:]
        st_al = (st // 8) * 8
        npairs = (en - st_al + 2 * C - 1) // (2 * C)
        return st_al, npairs

    s0, n0 = prep(rows0)
    s1, n1 = prep(rows1)
    meta = jnp.concatenate([s0, s1, n0, n1]).astype(jnp.int32)

    npad = EP - rows0.shape[0]
    padi = jnp.full((npad,), NPAD, jnp.int32)
    padz = jnp.zeros((npad,), jnp.float32)
    pad0 = jnp.zeros((npad,), jnp.int32)
    rows_s = jnp.concatenate([rows0, padi, rows1, padi])
    cols_s = jnp.concatenate([cols0, pad0, cols1, pad0])
    data_s = jnp.concatenate([data0, padz, data1, padz])
    kern_s = jnp.concatenate([kernel0, padz, kernel1, padz])

    partials = _sc_spmm(f, rows_s, cols_s, data_s, kern_s, meta)
    partials = partials.reshape(NCORES, NPAD, B)

    biasp = jnp.concatenate([bias, jnp.zeros((NPAD - N,), jnp.float32)])
    bias2d = biasp.reshape(NPAD // 128, 1, 128)

    out_full = _tc_combine(partials, bias2d)
    return out_full[:, :N]


# speedup vs baseline: 7.5521x; 2.6754x over previous
import dataclasses

import jax
import jax.numpy as jnp
from jax import lax
from jax.experimental import pallas as pl
from jax.experimental.pallas import tpu as pltpu
from jax.experimental.pallas import tpu_sc as plsc

N = 10000
B = 128
NPAD = 10240
NTILES = 16
NCORES = 2
R = NPAD // NTILES
C = 128
L = 16
EP = 320000 + 4 * C


def _scalar_from(ref, idx):
    v = plsc.load_gather(ref, [jnp.full((L,), idx, jnp.int32)])
    return jnp.max(v)


def _sc_body(f_hbm, rows_hbm, cols_hbm, data_hbm, kern_hbm, meta_hbm, out_hbm,
             meta_v, cols0_v, rows0_v, data0_v, kern0_v,
             cols1_v, rows1_v, data1_v, kern1_v,
             vals_v, rl_v, g0_v, g1_v, acc_v, se0, se1, sg0, sg1):
    c = lax.axis_index("c")
    s = lax.axis_index("s")
    wid = c * NTILES + s
    base = s * R

    pltpu.sync_copy(meta_hbm, meta_v)
    start = _scalar_from(meta_v, wid)
    npairs = _scalar_from(meta_v, wid + 32)

    iota = lax.iota(jnp.int32, L)
    col_idx = [iota + j * L for j in range(B // L)]
    zeros = jnp.zeros((L,), jnp.float32)

    @plsc.parallel_loop(0, R, unroll=4)
    def _(r):
        rfull = jnp.full((L,), r, jnp.int32)
        for j in range(B // L):
            plsc.store_scatter(acc_v, [rfull, col_idx[j]], zeros)

    ebase = c * EP
    bufs = ((cols0_v, rows0_v, data0_v, kern0_v, g0_v, se0, sg0),
            (cols1_v, rows1_v, data1_v, kern1_v, g1_v, se1, sg1))

    def edges_issue(n, half):
        cv, rv, dv, kv, _, se, _ = bufs[half]
        off = pl.multiple_of(ebase + start + n * C, 8)
        pltpu.async_copy(cols_hbm.at[pl.ds(off, C)], cv, se)
        pltpu.async_copy(rows_hbm.at[pl.ds(off, C)], rv, se)
        pltpu.async_copy(data_hbm.at[pl.ds(off, C)], dv, se)
        pltpu.async_copy(kern_hbm.at[pl.ds(off, C)], kv, se)

    def edges_drain(half):
        cv, rv, dv, kv, _, se, _ = bufs[half]
        pltpu.make_async_copy(cols_hbm.at[pl.ds(0, C)], cv, se).wait()
        pltpu.make_async_copy(rows_hbm.at[pl.ds(0, C)], rv, se).wait()
        pltpu.make_async_copy(data_hbm.at[pl.ds(0, C)], dv, se).wait()
        pltpu.make_async_copy(kern_hbm.at[pl.ds(0, C)], kv, se).wait()

    def gather_issue(half):
        cv, _, _, _, gv, _, sg = bufs[half]
        pltpu.async_copy(f_hbm.at[cv], gv, sg)

    def gather_drain(half):
        cv, _, _, _, gv, _, sg = bufs[half]
        pltpu.make_async_copy(f_hbm.at[cv], gv, sg).wait()

    edges_issue(0, 0)
    edges_drain(0)
    gather_issue(0)
    edges_issue(1, 1)

    def process(n, half):
        cv, rv, dv, kv, gv, _, _ = bufs[half]
        other = 1 - half
        edges_drain(other)
        gather_issue(other)
        for g in range(C // L):
            sl = pl.ds(g * L, L)
            r = rv[sl]
            val = dv[sl] * kv[sl]
            inr = (r >= base) & (r < base + R)
            vals_v[sl] = jnp.where(inr, val, 0.0)
            rl_v[sl] = jnp.clip(r - base, 0, R - 1)
        gather_drain(half)
        edges_issue(n + 2, half)
        @plsc.parallel_loop(0, C, unroll=4)
        def _(i):
            ii = jnp.full((L,), i, jnp.int32)
            vspl = plsc.load_gather(vals_v, [ii])
            rspl = plsc.load_gather(rl_v, [ii])
            for j in range(B // L):
                gvv = plsc.load_gather(gv, [ii, col_idx[j]])
                plsc.addupdate_scatter(acc_v, [rspl, col_idx[j]], gvv * vspl)

    @pl.loop(0, npairs)
    def _(p):
        process(2 * p, 0)
        process(2 * p + 1, 1)

    gather_drain(0)
    edges_drain(1)

    obase = pl.multiple_of(c * NPAD + base, 8)
    pltpu.sync_copy(acc_v, out_hbm.at[pl.ds(obase, R)])


def _sc_spmm(f, rows_s, cols_s, data_s, kern_s, meta):
    mesh = plsc.VectorSubcoreMesh(core_axis_name="c", subcore_axis_name="s")
    cp = pltpu.CompilerParams()
    if "needs_layout_passes" in pltpu.CompilerParams.__dataclass_fields__:
        cp = dataclasses.replace(cp, needs_layout_passes=False)
    kfn = pl.kernel(
        _sc_body,
        compiler_params=cp,
        out_type=jax.ShapeDtypeStruct((NCORES * NPAD, B), jnp.float32),
        mesh=mesh,
        scratch_types=[
            pltpu.VMEM((64,), jnp.int32),
            pltpu.VMEM((C,), jnp.int32),
            pltpu.VMEM((C,), jnp.int32),
            pltpu.VMEM((C,), jnp.float32),
            pltpu.VMEM((C,), jnp.float32),
            pltpu.VMEM((C,), jnp.int32),
            pltpu.VMEM((C,), jnp.int32),
            pltpu.VMEM((C,), jnp.float32),
            pltpu.VMEM((C,), jnp.float32),
            pltpu.VMEM((C,), jnp.float32),
            pltpu.VMEM((C,), jnp.int32),
            pltpu.VMEM((C, B), jnp.float32),
            pltpu.VMEM((C, B), jnp.float32),
            pltpu.VMEM((R, B), jnp.float32),
            pltpu.SemaphoreType.DMA,
            pltpu.SemaphoreType.DMA,
            pltpu.SemaphoreType.DMA,
            pltpu.SemaphoreType.DMA,
        ],
    )
    return kfn(f, rows_s, cols_s, data_s, kern_s, meta)


def _combine_body(p_ref, b_ref, o_ref):
    t = p_ref[0] + p_ref[1]
    o_ref[...] = jnp.maximum(t.T + b_ref[0], 0.0)


def _tc_combine(partials, bias2d):
    return pl.pallas_call(
        _combine_body,
        grid=(NPAD // 128,),
        in_specs=[
            pl.BlockSpec((2, 128, 128), lambda i: (0, i, 0)),
            pl.BlockSpec((1, 1, 128), lambda i: (i, 0, 0)),
        ],
        out_specs=pl.BlockSpec((128, 128), lambda i: (0, i)),
        out_shape=jax.ShapeDtypeStruct((B, NPAD), jnp.float32),
    )(partials, bias2d)


def kernel(features, rows0, cols0, data0, kernel0, rows1, cols1, data1,
           kernel1, bias):
    f = features.T

    bounds = jnp.arange(NTILES + 1, dtype=jnp.int32) * R

    def prep(rows):
        ss = jnp.searchsorted(rows, bounds).astype(jnp.int32)
        st = ss[:NTILES]
        en = ss[1:]
        st_al = (st // 8) * 8
        npairs = (en - st_al + 2 * C - 1) // (2 * C)
        return st_al, npairs

    s0, n0 = prep(rows0)
    s1, n1 = prep(rows1)
    meta = jnp.concatenate([s0, s1, n0, n1]).astype(jnp.int32)

    npad = EP - rows0.shape[0]
    padi = jnp.full((npad,), NPAD, jnp.int32)
    padz = jnp.zeros((npad,), jnp.float32)
    pad0 = jnp.zeros((npad,), jnp.int32)
    rows_s = jnp.concatenate([rows0, padi, rows1, padi])
    cols_s = jnp.concatenate([cols0, pad0, cols1, pad0])
    data_s = jnp.concatenate([data0, padz, data1, padz])
    kern_s = jnp.concatenate([kernel0, padz, kernel1, padz])

    partials = _sc_spmm(f, rows_s, cols_s, data_s, kern_s, meta)
    partials = partials.reshape(NCORES, NPAD, B)

    biasp = jnp.concatenate([bias, jnp.zeros((NPAD - N,), jnp.float32)])
    bias2d = biasp.reshape(NPAD // 128, 1, 128)

    out_full = _tc_combine(partials, bias2d)
    return out_full[:, :N]
